# Initial kernel scaffold; baseline (speedup 1.0000x reference)
import jax
import jax.numpy as jnp
from jax.experimental import pallas as pl


def kernel(h, positions, edge_index, We1, be1, We2, be2, Wn1, bn1, Wn2, bn2):
    raise NotImplementedError("write your pallas kernel here")


# speedup vs baseline: 2.5470x; 2.5470x over previous
import functools

import jax
import jax.numpy as jnp
from jax import lax
from jax.experimental import pallas as pl
from jax.experimental.pallas import tpu as pltpu
from jax.experimental.pallas import tpu_sc as plsc

N = 10000
E = 320000
D = 128
H = 128

NC, NS, L = 2, 16, 16
NW = NC * NS
EPW = E // NW
C = 80
CHUNKS = EPW // C
GROUPS = C // L
ZB = N // NS // 5

_mesh = plsc.VectorSubcoreMesh(core_axis_name="c", subcore_axis_name="s")


def _pre_body(h_ref, wa_ref, wb_ref, be1_ref, a_ref, b_ref):
    x = h_ref[...]
    a_ref[...] = jnp.dot(x, wa_ref[...], preferred_element_type=jnp.float32)
    b_ref[...] = (jnp.dot(x, wb_ref[...], preferred_element_type=jnp.float32)
                  + be1_ref[...])


def _precompute(h, We1a, We1b, be1):
    BN = 400
    return pl.pallas_call(
        _pre_body,
        grid=(N // BN,),
        in_specs=[
            pl.BlockSpec((BN, D), lambda i: (i, 0)),
            pl.BlockSpec((D, H), lambda i: (0, 0)),
            pl.BlockSpec((D, H), lambda i: (0, 0)),
            pl.BlockSpec((1, H), lambda i: (0, 0)),
        ],
        out_specs=[
            pl.BlockSpec((BN, H), lambda i: (i, 0)),
            pl.BlockSpec((BN, H), lambda i: (i, 0)),
        ],
        out_shape=[
            jax.ShapeDtypeStruct((N, H), jnp.float32),
            jax.ShapeDtypeStruct((N, H), jnp.float32),
        ],
    )(h, We1a, We1b, be1)


@functools.partial(
    pl.kernel,
    out_type=jax.ShapeDtypeStruct((E, H), jnp.float32),
    mesh=_mesh,
    scratch_types=[
        pltpu.VMEM((C,), jnp.int32),
        pltpu.VMEM((C,), jnp.int32),
        pltpu.VMEM((C,), jnp.float32),
        pltpu.VMEM((C,), jnp.float32),
        pltpu.VMEM((C,), jnp.float32),
        pltpu.VMEM((C,), jnp.float32),
        pltpu.VMEM((C,), jnp.float32),
        pltpu.VMEM((C,), jnp.float32),
        pltpu.VMEM((C, H), jnp.float32),
        pltpu.VMEM((C, H), jnp.float32),
        pltpu.VMEM((C + L,), jnp.float32),
        pltpu.VMEM((H,), jnp.float32),
        pltpu.SemaphoreType.DMA,
        pltpu.SemaphoreType.DMA,
    ],
)
def _sc_edge(row, col, A, B, px, py, pz, wr, t_out,
             row_v, col_v, pxr_v, pyr_v, pzr_v, pxc_v, pyc_v, pzc_v,
             a_v, b_v, rad_v, wr_v, s1, s2):
    wid = lax.axis_index("s") * NC + lax.axis_index("c")
    base0 = wid * EPW
    pltpu.sync_copy(wr, wr_v)

    def chunk(ci, carry):
        base = base0 + ci * C
        pltpu.sync_copy(row.at[pl.ds(base, C)], row_v)
        pltpu.sync_copy(col.at[pl.ds(base, C)], col_v)
        cp1 = pltpu.async_copy(A.at[row_v], a_v, s1)
        cp2 = pltpu.async_copy(B.at[col_v], b_v, s1)
        g1 = pltpu.async_copy(px.at[row_v], pxr_v, s2)
        g2 = pltpu.async_copy(py.at[row_v], pyr_v, s2)
        g3 = pltpu.async_copy(pz.at[row_v], pzr_v, s2)
        g4 = pltpu.async_copy(px.at[col_v], pxc_v, s2)
        g5 = pltpu.async_copy(py.at[col_v], pyc_v, s2)
        g6 = pltpu.async_copy(pz.at[col_v], pzc_v, s2)
        g1.wait(); g2.wait(); g3.wait(); g4.wait(); g5.wait(); g6.wait()
        for g in range(GROUPS):
            sl = pl.ds(g * L, L)
            dx = pxr_v[sl] - pxc_v[sl]
            dy = pyr_v[sl] - pyc_v[sl]
            dz = pzr_v[sl] - pzc_v[sl]
            rad_v[sl] = dx * dx + dy * dy + dz * dz
        cp1.wait()
        cp2.wait()

        def edge_body(e, carry2):
            rv = rad_v[pl.ds(e, L)]
            sp = jnp.full((L,), rv[0], jnp.float32)
            for f in range(H // L):
                sl = pl.ds(f * L, L)
                a_v[e, sl] = a_v[e, sl] + b_v[e, sl] + sp * wr_v[sl]
            return carry2

        lax.fori_loop(0, C, edge_body, 0, unroll=2)
        pltpu.sync_copy(a_v, t_out.at[pl.ds(base, C)])
        return carry

    lax.fori_loop(0, CHUNKS, chunk, 0)


def _edge_mlp_body(t_ref, w2_ref, be2_ref, o_ref):
    t = jnp.maximum(t_ref[...], 0.0)
    u = jnp.dot(t, w2_ref[...], preferred_element_type=jnp.float32)
    o_ref[...] = jnp.maximum(u + be2_ref[...], 0.0)


def _edge_mlp(t_pre, We2, be2):
    BE = 512
    return pl.pallas_call(
        _edge_mlp_body,
        grid=(E // BE,),
        in_specs=[
            pl.BlockSpec((BE, H), lambda i: (i, 0)),
            pl.BlockSpec((H, H), lambda i: (0, 0)),
            pl.BlockSpec((1, H), lambda i: (0, 0)),
        ],
        out_specs=pl.BlockSpec((BE, H), lambda i: (i, 0)),
        out_shape=jax.ShapeDtypeStruct((E, H), jnp.float32),
    )(t_pre, We2, be2)


@functools.partial(
    pl.kernel,
    out_type=jax.ShapeDtypeStruct((NC, N, H), jnp.float32),
    mesh=_mesh,
    scratch_types=[
        pltpu.VMEM((C,), jnp.int32),
        pltpu.VMEM((C, H), jnp.float32),
        pltpu.VMEM((ZB, H), jnp.float32),
        pltpu.VMEM_SHARED((N, H), jnp.float32),
        pltpu.SemaphoreType.DMA,
    ],
)
def _sc_scatter(row, ef, out, row_v, ef_v, z_v, acc, sem):
    cid = lax.axis_index("c")
    sid = lax.axis_index("s")
    wid = sid * NC + cid
    base0 = wid * EPW
    zeros = jnp.zeros((L,), jnp.float32)

    def zrow(r, carry):
        for f in range(H // L):
            z_v[r, pl.ds(f * L, L)] = zeros
        return carry

    lax.fori_loop(0, ZB, zrow, 0)
    rows_per_tile = N // NS
    for j in range(rows_per_tile // ZB):
        pltpu.sync_copy(z_v, acc.at[pl.ds(sid * rows_per_tile + j * ZB, ZB)])
    plsc.subcore_barrier()

    def chunk(ci, carry):
        base = base0 + ci * C
        pltpu.sync_copy(row.at[pl.ds(base, C)], row_v)
        pltpu.sync_copy(ef.at[pl.ds(base, C)], ef_v)
        pltpu.sync_copy(ef_v, acc.at[row_v], add=True)
        return carry

    lax.fori_loop(0, CHUNKS, chunk, 0)
    plsc.subcore_barrier()
    RPT = 624
    start = sid * RPT
    pltpu.sync_copy(acc.at[pl.ds(start, RPT)], out.at[cid, pl.ds(start, RPT)])

    @pl.when(sid == NS - 1)
    def _tail():
        pltpu.sync_copy(acc.at[pl.ds(NS * RPT, N - NS * RPT)],
                        out.at[cid, pl.ds(NS * RPT, N - NS * RPT)])


def _node_body(h_ref, agg_ref, w1h_ref, w1a_ref, bn1_ref, w2_ref, bn2_ref,
               o_ref):
    hx = h_ref[...]
    agg = agg_ref[0] + agg_ref[1]
    u = (jnp.dot(hx, w1h_ref[...], preferred_element_type=jnp.float32)
         + jnp.dot(agg, w1a_ref[...], preferred_element_type=jnp.float32)
         + bn1_ref[...])
    u = jnp.maximum(u, 0.0)
    o_ref[...] = (jnp.dot(u, w2_ref[...], preferred_element_type=jnp.float32)
                  + bn2_ref[...] + hx)


def _node_mlp(h, agg2, Wn1h, Wn1a, bn1, Wn2, bn2):
    BN = 400
    return pl.pallas_call(
        _node_body,
        grid=(N // BN,),
        in_specs=[
            pl.BlockSpec((BN, D), lambda i: (i, 0)),
            pl.BlockSpec((NC, BN, H), lambda i: (0, i, 0)),
            pl.BlockSpec((D, H), lambda i: (0, 0)),
            pl.BlockSpec((H, H), lambda i: (0, 0)),
            pl.BlockSpec((1, H), lambda i: (0, 0)),
            pl.BlockSpec((H, D), lambda i: (0, 0)),
            pl.BlockSpec((1, D), lambda i: (0, 0)),
        ],
        out_specs=pl.BlockSpec((BN, D), lambda i: (i, 0)),
        out_shape=jax.ShapeDtypeStruct((N, D), jnp.float32),
    )(h, agg2, Wn1h, Wn1a, bn1, Wn2, bn2)


def kernel(h, positions, edge_index, We1, be1, We2, be2, Wn1, bn1, Wn2, bn2):
    We1a = We1[:D]
    We1b = We1[D:2 * D]
    wr = We1[2 * D]
    px = positions[:, 0]
    py = positions[:, 1]
    pz = positions[:, 2]
    A, B = _precompute(h, We1a, We1b, be1.reshape(1, H))
    row = edge_index[0]
    col = edge_index[1]
    t_pre = _sc_edge(row, col, A, B, px, py, pz, wr)
    ef = _edge_mlp(t_pre, We2, be2.reshape(1, H))
    agg2 = _sc_scatter(row, ef)
    h_out = _node_mlp(h, agg2, Wn1[:D], Wn1[D:], bn1.reshape(1, H), Wn2,
                      bn2.reshape(1, D))
    return (h_out, positions)


# speedup vs baseline: 3.2566x; 1.2786x over previous
import functools

import jax
import jax.numpy as jnp
from jax import lax
from jax.experimental import pallas as pl
from jax.experimental.pallas import tpu as pltpu
from jax.experimental.pallas import tpu_sc as plsc

N = 10000
E = 320000
D = 128
H = 128

NC, NS, L = 2, 16, 16
NW = NC * NS
EPW = E // NW
C = 80
CHUNKS = EPW // C
GROUPS = C // L
ZB = N // NS // 5

_mesh = plsc.VectorSubcoreMesh(core_axis_name="c", subcore_axis_name="s")


def _pre_body(h_ref, wa_ref, wb_ref, be1_ref, a_ref, b_ref):
    x = h_ref[...]
    a_ref[...] = jnp.dot(x, wa_ref[...], preferred_element_type=jnp.float32)
    b_ref[...] = (jnp.dot(x, wb_ref[...], preferred_element_type=jnp.float32)
                  + be1_ref[...])


def _precompute(h, We1a, We1b, be1):
    BN = 400
    return pl.pallas_call(
        _pre_body,
        grid=(N // BN,),
        in_specs=[
            pl.BlockSpec((BN, D), lambda i: (i, 0)),
            pl.BlockSpec((D, H), lambda i: (0, 0)),
            pl.BlockSpec((D, H), lambda i: (0, 0)),
            pl.BlockSpec((1, H), lambda i: (0, 0)),
        ],
        out_specs=[
            pl.BlockSpec((BN, H), lambda i: (i, 0)),
            pl.BlockSpec((BN, H), lambda i: (i, 0)),
        ],
        out_shape=[
            jax.ShapeDtypeStruct((N, H), jnp.float32),
            jax.ShapeDtypeStruct((N, H), jnp.float32),
        ],
    )(h, We1a, We1b, be1)


@functools.partial(
    pl.kernel,
    out_type=jax.ShapeDtypeStruct((E, H), jnp.float32),
    mesh=_mesh,
    scratch_types=[
        [pltpu.VMEM((C,), jnp.int32)] * 2,
        [pltpu.VMEM((C,), jnp.int32)] * 2,
        [pltpu.VMEM((C,), jnp.float32)] * 6,
        [pltpu.VMEM((C,), jnp.float32)] * 6,
        [pltpu.VMEM((C, H), jnp.float32)] * 2,
        [pltpu.VMEM((C, H), jnp.float32)] * 2,
        pltpu.VMEM((C + L,), jnp.float32),
        pltpu.VMEM((H,), jnp.float32),
        [pltpu.SemaphoreType.DMA] * 2,
        [pltpu.SemaphoreType.DMA] * 2,
        [pltpu.SemaphoreType.DMA] * 2,
    ],
)
def _sc_edge(row, col, A, B, px, py, pz, wr, t_out,
             row_v, col_v, pos0, pos1, a_v, b_v, rad_v, wr_v,
             s_ab, s_pos, s_out):
    wid = lax.axis_index("s") * NC + lax.axis_index("c")
    base0 = wid * EPW
    pltpu.sync_copy(wr, wr_v)
    wvecs = [wr_v[pl.ds(f * L, L)] for f in range(H // L)]
    pos = (pos0, pos1)

    def issue(ci, sl):
        base = base0 + ci * C
        pltpu.sync_copy(row.at[pl.ds(base, C)], row_v[sl])
        pltpu.sync_copy(col.at[pl.ds(base, C)], col_v[sl])
        pltpu.async_copy(A.at[row_v[sl]], a_v[sl], s_ab[sl])
        pltpu.async_copy(B.at[col_v[sl]], b_v[sl], s_ab[sl])
        pltpu.async_copy(px.at[row_v[sl]], pos[sl][0], s_pos[sl])
        pltpu.async_copy(py.at[row_v[sl]], pos[sl][1], s_pos[sl])
        pltpu.async_copy(pz.at[row_v[sl]], pos[sl][2], s_pos[sl])
        pltpu.async_copy(px.at[col_v[sl]], pos[sl][3], s_pos[sl])
        pltpu.async_copy(py.at[col_v[sl]], pos[sl][4], s_pos[sl])
        pltpu.async_copy(pz.at[col_v[sl]], pos[sl][5], s_pos[sl])

    def wait_gathers(sl):
        pltpu.make_async_copy(A.at[row_v[sl]], a_v[sl], s_ab[sl]).wait()
        pltpu.make_async_copy(B.at[col_v[sl]], b_v[sl], s_ab[sl]).wait()
        for j in range(6):
            pltpu.make_async_copy(px.at[row_v[sl]], pos[sl][j],
                                  s_pos[sl]).wait()

    def wait_out(ci, sl):
        base = base0 + ci * C
        pltpu.make_async_copy(a_v[sl], t_out.at[pl.ds(base, C)],
                              s_out[sl]).wait()

    issue(0, 0)

    def iteration(i, sl):
        q = 1 - sl

        @pl.when(i < CHUNKS - 1)
        def _prefetch():
            @pl.when(i >= 1)
            def _():
                wait_out(i - 1, q)

            issue(i + 1, q)

        wait_gathers(sl)
        for g in range(GROUPS):
            gsl = pl.ds(g * L, L)
            dx = pos[sl][0][gsl] - pos[sl][3][gsl]
            dy = pos[sl][1][gsl] - pos[sl][4][gsl]
            dz = pos[sl][2][gsl] - pos[sl][5][gsl]
            rad_v[gsl] = dx * dx + dy * dy + dz * dz

        def edge_body(e, carry2):
            rv = rad_v[pl.ds(e, L)]
            sp = jnp.full((L,), rv[0], jnp.float32)
            for f in range(H // L):
                sl2 = pl.ds(f * L, L)
                a_v[sl][e, sl2] = (a_v[sl][e, sl2] + b_v[sl][e, sl2]
                                   + sp * wvecs[f])
            return carry2

        lax.fori_loop(0, C, edge_body, 0, unroll=2)
        base = base0 + i * C
        pltpu.async_copy(a_v[sl], t_out.at[pl.ds(base, C)], s_out[sl])

    def chunk(i, carry):
        p = lax.rem(i, 2)

        @pl.when(p == 0)
        def _s0():
            iteration(i, 0)

        @pl.when(p == 1)
        def _s1():
            iteration(i, 1)

        return carry

    lax.fori_loop(0, CHUNKS, chunk, 0)
    wait_out(CHUNKS - 2, (CHUNKS - 2) % 2)
    wait_out(CHUNKS - 1, (CHUNKS - 1) % 2)


def _edge_mlp_body(t_ref, w2_ref, be2_ref, o_ref):
    t = jnp.maximum(t_ref[...], 0.0)
    u = jnp.dot(t, w2_ref[...], preferred_element_type=jnp.float32)
    o_ref[...] = jnp.maximum(u + be2_ref[...], 0.0)


def _edge_mlp(t_pre, We2, be2):
    BE = 512
    return pl.pallas_call(
        _edge_mlp_body,
        grid=(E // BE,),
        in_specs=[
            pl.BlockSpec((BE, H), lambda i: (i, 0)),
            pl.BlockSpec((H, H), lambda i: (0, 0)),
            pl.BlockSpec((1, H), lambda i: (0, 0)),
        ],
        out_specs=pl.BlockSpec((BE, H), lambda i: (i, 0)),
        out_shape=jax.ShapeDtypeStruct((E, H), jnp.float32),
    )(t_pre, We2, be2)


@functools.partial(
    pl.kernel,
    out_type=jax.ShapeDtypeStruct((NC, N, H), jnp.float32),
    mesh=_mesh,
    scratch_types=[
        [pltpu.VMEM((C,), jnp.int32)] * 2,
        [pltpu.VMEM((C, H), jnp.float32)] * 2,
        pltpu.VMEM((ZB, H), jnp.float32),
        pltpu.VMEM_SHARED((N, H), jnp.float32),
        [pltpu.SemaphoreType.DMA] * 2,
        [pltpu.SemaphoreType.DMA] * 2,
    ],
)
def _sc_scatter(row, ef, out, row_v, ef_v, z_v, acc, s_ld, s_sc):
    cid = lax.axis_index("c")
    sid = lax.axis_index("s")
    wid = sid * NC + cid
    base0 = wid * EPW
    zeros = jnp.zeros((L,), jnp.float32)

    def zrow(r, carry):
        for f in range(H // L):
            z_v[r, pl.ds(f * L, L)] = zeros
        return carry

    lax.fori_loop(0, ZB, zrow, 0)
    rows_per_tile = N // NS
    for j in range(rows_per_tile // ZB):
        pltpu.sync_copy(z_v, acc.at[pl.ds(sid * rows_per_tile + j * ZB, ZB)])
    plsc.subcore_barrier()

    def issue(ci, sl):
        base = base0 + ci * C
        pltpu.sync_copy(row.at[pl.ds(base, C)], row_v[sl])
        pltpu.async_copy(ef.at[pl.ds(base, C)], ef_v[sl], s_ld[sl])

    def wait_ld(ci, sl):
        base = base0 + ci * C
        pltpu.make_async_copy(ef.at[pl.ds(base, C)], ef_v[sl],
                              s_ld[sl]).wait()

    def wait_sc(sl):
        pltpu.make_async_copy(ef_v[sl], acc.at[row_v[sl]], s_sc[sl]).wait()

    issue(0, 0)

    def iteration(i, sl):
        q = 1 - sl

        @pl.when(i < CHUNKS - 1)
        def _prefetch():
            @pl.when(i >= 2)
            def _():
                wait_sc(q)

            issue(i + 1, q)

        wait_ld(i, sl)
        pltpu.async_copy(ef_v[sl], acc.at[row_v[sl]], s_sc[sl], add=True)

    def chunk(i, carry):
        p = lax.rem(i, 2)

        @pl.when(p == 0)
        def _s0():
            iteration(i, 0)

        @pl.when(p == 1)
        def _s1():
            iteration(i, 1)

        return carry

    lax.fori_loop(0, CHUNKS, chunk, 0)
    wait_sc(0)
    wait_sc(1)
    plsc.subcore_barrier()
    RPT = 624
    start = sid * RPT
    pltpu.sync_copy(acc.at[pl.ds(start, RPT)], out.at[cid, pl.ds(start, RPT)])

    @pl.when(sid == NS - 1)
    def _tail():
        pltpu.sync_copy(acc.at[pl.ds(NS * RPT, N - NS * RPT)],
                        out.at[cid, pl.ds(NS * RPT, N - NS * RPT)])


def _node_body(h_ref, agg_ref, w1h_ref, w1a_ref, bn1_ref, w2_ref, bn2_ref,
               o_ref):
    hx = h_ref[...]
    agg = agg_ref[0] + agg_ref[1]
    u = (jnp.dot(hx, w1h_ref[...], preferred_element_type=jnp.float32)
         + jnp.dot(agg, w1a_ref[...], preferred_element_type=jnp.float32)
         + bn1_ref[...])
    u = jnp.maximum(u, 0.0)
    o_ref[...] = (jnp.dot(u, w2_ref[...], preferred_element_type=jnp.float32)
                  + bn2_ref[...] + hx)


def _node_mlp(h, agg2, Wn1h, Wn1a, bn1, Wn2, bn2):
    BN = 400
    return pl.pallas_call(
        _node_body,
        grid=(N // BN,),
        in_specs=[
            pl.BlockSpec((BN, D), lambda i: (i, 0)),
            pl.BlockSpec((NC, BN, H), lambda i: (0, i, 0)),
            pl.BlockSpec((D, H), lambda i: (0, 0)),
            pl.BlockSpec((H, H), lambda i: (0, 0)),
            pl.BlockSpec((1, H), lambda i: (0, 0)),
            pl.BlockSpec((H, D), lambda i: (0, 0)),
            pl.BlockSpec((1, D), lambda i: (0, 0)),
        ],
        out_specs=pl.BlockSpec((BN, D), lambda i: (i, 0)),
        out_shape=jax.ShapeDtypeStruct((N, D), jnp.float32),
    )(h, agg2, Wn1h, Wn1a, bn1, Wn2, bn2)


def kernel(h, positions, edge_index, We1, be1, We2, be2, Wn1, bn1, Wn2, bn2):
    We1a = We1[:D]
    We1b = We1[D:2 * D]
    wr = We1[2 * D]
    px = positions[:, 0]
    py = positions[:, 1]
    pz = positions[:, 2]
    A, B = _precompute(h, We1a, We1b, be1.reshape(1, H))
    row = edge_index[0]
    col = edge_index[1]
    t_pre = _sc_edge(row, col, A, B, px, py, pz, wr)
    ef = _edge_mlp(t_pre, We2, be2.reshape(1, H))
    agg2 = _sc_scatter(row, ef)
    h_out = _node_mlp(h, agg2, Wn1[:D], Wn1[D:], bn1.reshape(1, H), Wn2,
                      bn2.reshape(1, D))
    return (h_out, positions)


# speedup vs baseline: 4.4001x; 1.3511x over previous
import functools

import jax
import jax.numpy as jnp
from jax import lax
from jax.experimental import pallas as pl
from jax.experimental.pallas import tpu as pltpu
from jax.experimental.pallas import tpu_sc as plsc

N = 10000
E = 320000
D = 128
H = 128

NC, NS, L = 2, 16, 16
NW = NC * NS
EPW = E // NW
C = 80
CHUNKS = EPW // C
GROUPS = C // L
ZB = N // NS // 5

_mesh = plsc.VectorSubcoreMesh(core_axis_name="c", subcore_axis_name="s")


def _pre_body(h_ref, wa_ref, wb_ref, be1_ref, a_ref, b_ref):
    x = h_ref[...]
    a_ref[...] = jnp.dot(x, wa_ref[...], preferred_element_type=jnp.float32)
    b_ref[...] = (jnp.dot(x, wb_ref[...], preferred_element_type=jnp.float32)
                  + be1_ref[...])


def _precompute(h, We1a, We1b, be1):
    BN = 400
    return pl.pallas_call(
        _pre_body,
        grid=(N // BN,),
        in_specs=[
            pl.BlockSpec((BN, D), lambda i: (i, 0)),
            pl.BlockSpec((D, H), lambda i: (0, 0)),
            pl.BlockSpec((D, H), lambda i: (0, 0)),
            pl.BlockSpec((1, H), lambda i: (0, 0)),
        ],
        out_specs=[
            pl.BlockSpec((BN, H), lambda i: (i, 0)),
            pl.BlockSpec((BN, H), lambda i: (i, 0)),
        ],
        out_shape=[
            jax.ShapeDtypeStruct((N, H), jnp.float32),
            jax.ShapeDtypeStruct((N, H), jnp.float32),
        ],
    )(h, We1a, We1b, be1)


@functools.partial(
    pl.kernel,
    out_type=jax.ShapeDtypeStruct((E, H), jnp.float32),
    mesh=_mesh,
    scratch_types=[
        [pltpu.VMEM((C,), jnp.int32)] * 2,
        [pltpu.VMEM((C,), jnp.int32)] * 2,
        [pltpu.VMEM((C,), jnp.float32)] * 6,
        [pltpu.VMEM((C,), jnp.float32)] * 6,
        [pltpu.VMEM((C, H), jnp.float32)] * 2,
        [pltpu.VMEM((C, H), jnp.float32)] * 2,
        pltpu.VMEM((C + L,), jnp.float32),
        pltpu.VMEM((H,), jnp.float32),
        [pltpu.SemaphoreType.DMA] * 2,
        [pltpu.SemaphoreType.DMA] * 2,
        [pltpu.SemaphoreType.DMA] * 2,
    ],
)
def _sc_edge(row, col, A, B, px, py, pz, wr, t_out,
             row_v, col_v, pos0, pos1, a_v, b_v, rad_v, wr_v,
             s_ab, s_pos, s_out):
    wid = lax.axis_index("s") * NC + lax.axis_index("c")
    base0 = wid * EPW
    pltpu.sync_copy(wr, wr_v)
    wvecs = [wr_v[pl.ds(f * L, L)] for f in range(H // L)]
    pos = (pos0, pos1)

    def issue(ci, sl):
        base = base0 + ci * C
        pltpu.sync_copy(row.at[pl.ds(base, C)], row_v[sl])
        pltpu.sync_copy(col.at[pl.ds(base, C)], col_v[sl])
        pltpu.async_copy(A.at[row_v[sl]], a_v[sl], s_ab[sl])
        pltpu.async_copy(B.at[col_v[sl]], b_v[sl], s_ab[sl])
        pltpu.async_copy(px.at[row_v[sl]], pos[sl][0], s_pos[sl])
        pltpu.async_copy(py.at[row_v[sl]], pos[sl][1], s_pos[sl])
        pltpu.async_copy(pz.at[row_v[sl]], pos[sl][2], s_pos[sl])
        pltpu.async_copy(px.at[col_v[sl]], pos[sl][3], s_pos[sl])
        pltpu.async_copy(py.at[col_v[sl]], pos[sl][4], s_pos[sl])
        pltpu.async_copy(pz.at[col_v[sl]], pos[sl][5], s_pos[sl])

    def wait_gathers(sl):
        pltpu.make_async_copy(A.at[row_v[sl]], a_v[sl], s_ab[sl]).wait()
        pltpu.make_async_copy(B.at[col_v[sl]], b_v[sl], s_ab[sl]).wait()
        for j in range(6):
            pltpu.make_async_copy(px.at[row_v[sl]], pos[sl][j],
                                  s_pos[sl]).wait()

    def wait_out(ci, sl):
        base = base0 + ci * C
        pltpu.make_async_copy(a_v[sl], t_out.at[pl.ds(base, C)],
                              s_out[sl]).wait()

    issue(0, 0)

    def iteration(i, sl):
        q = 1 - sl

        @pl.when(i < CHUNKS - 1)
        def _prefetch():
            @pl.when(i >= 1)
            def _():
                wait_out(i - 1, q)

            issue(i + 1, q)

        wait_gathers(sl)
        for g in range(GROUPS):
            gsl = pl.ds(g * L, L)
            dx = pos[sl][0][gsl] - pos[sl][3][gsl]
            dy = pos[sl][1][gsl] - pos[sl][4][gsl]
            dz = pos[sl][2][gsl] - pos[sl][5][gsl]
            rad_v[gsl] = dx * dx + dy * dy + dz * dz

        def edge_body(e, carry2):
            rv = rad_v[pl.ds(e, L)]
            sp = jnp.full((L,), rv[0], jnp.float32)
            for f in range(H // L):
                sl2 = pl.ds(f * L, L)
                plsc.addupdate(a_v[sl].at[e, sl2],
                               b_v[sl][e, sl2] + sp * wvecs[f])
            return carry2

        lax.fori_loop(0, C, edge_body, 0, unroll=4)
        base = base0 + i * C
        pltpu.async_copy(a_v[sl], t_out.at[pl.ds(base, C)], s_out[sl])

    def chunk(i, carry):
        p = lax.rem(i, 2)

        @pl.when(p == 0)
        def _s0():
            iteration(i, 0)

        @pl.when(p == 1)
        def _s1():
            iteration(i, 1)

        return carry

    lax.fori_loop(0, CHUNKS, chunk, 0)
    wait_out(CHUNKS - 2, (CHUNKS - 2) % 2)
    wait_out(CHUNKS - 1, (CHUNKS - 1) % 2)


def _edge_mlp_body(t_ref, w2_ref, be2_ref, o_ref):
    t = jnp.maximum(t_ref[...], 0.0)
    u = jnp.dot(t, w2_ref[...], preferred_element_type=jnp.float32)
    o_ref[...] = jnp.maximum(u + be2_ref[...], 0.0)


def _edge_mlp(t_pre, We2, be2):
    BE = 512
    return pl.pallas_call(
        _edge_mlp_body,
        grid=(E // BE,),
        in_specs=[
            pl.BlockSpec((BE, H), lambda i: (i, 0)),
            pl.BlockSpec((H, H), lambda i: (0, 0)),
            pl.BlockSpec((1, H), lambda i: (0, 0)),
        ],
        out_specs=pl.BlockSpec((BE, H), lambda i: (i, 0)),
        out_shape=jax.ShapeDtypeStruct((E, H), jnp.float32),
    )(t_pre, We2, be2)


@functools.partial(
    pl.kernel,
    out_type=jax.ShapeDtypeStruct((NC, N, H), jnp.float32),
    mesh=_mesh,
    scratch_types=[
        [pltpu.VMEM((C,), jnp.int32)] * 2,
        [pltpu.VMEM((C, H), jnp.float32)] * 2,
        pltpu.VMEM((ZB, H), jnp.float32),
        pltpu.VMEM_SHARED((N, H), jnp.float32),
        [pltpu.SemaphoreType.DMA] * 2,
        [pltpu.SemaphoreType.DMA] * 2,
    ],
)
def _sc_scatter(row, ef, out, row_v, ef_v, z_v, acc, s_ld, s_sc):
    cid = lax.axis_index("c")
    sid = lax.axis_index("s")
    wid = sid * NC + cid
    base0 = wid * EPW
    zeros = jnp.zeros((L,), jnp.float32)

    def zrow(r, carry):
        for f in range(H // L):
            z_v[r, pl.ds(f * L, L)] = zeros
        return carry

    lax.fori_loop(0, ZB, zrow, 0)
    rows_per_tile = N // NS
    for j in range(rows_per_tile // ZB):
        pltpu.sync_copy(z_v, acc.at[pl.ds(sid * rows_per_tile + j * ZB, ZB)])
    plsc.subcore_barrier()

    def issue(ci, sl):
        base = base0 + ci * C
        pltpu.sync_copy(row.at[pl.ds(base, C)], row_v[sl])
        pltpu.async_copy(ef.at[pl.ds(base, C)], ef_v[sl], s_ld[sl])

    def wait_ld(ci, sl):
        base = base0 + ci * C
        pltpu.make_async_copy(ef.at[pl.ds(base, C)], ef_v[sl],
                              s_ld[sl]).wait()

    def wait_sc(sl):
        pltpu.make_async_copy(ef_v[sl], acc.at[row_v[sl]], s_sc[sl]).wait()

    issue(0, 0)

    def iteration(i, sl):
        q = 1 - sl

        @pl.when(i < CHUNKS - 1)
        def _prefetch():
            @pl.when(i >= 2)
            def _():
                wait_sc(q)

            issue(i + 1, q)

        wait_ld(i, sl)
        pltpu.async_copy(ef_v[sl], acc.at[row_v[sl]], s_sc[sl], add=True)

    def chunk(i, carry):
        p = lax.rem(i, 2)

        @pl.when(p == 0)
        def _s0():
            iteration(i, 0)

        @pl.when(p == 1)
        def _s1():
            iteration(i, 1)

        return carry

    lax.fori_loop(0, CHUNKS, chunk, 0)
    wait_sc(0)
    wait_sc(1)
    plsc.subcore_barrier()
    RPT = 624
    start = sid * RPT
    pltpu.sync_copy(acc.at[pl.ds(start, RPT)], out.at[cid, pl.ds(start, RPT)])

    @pl.when(sid == NS - 1)
    def _tail():
        pltpu.sync_copy(acc.at[pl.ds(NS * RPT, N - NS * RPT)],
                        out.at[cid, pl.ds(NS * RPT, N - NS * RPT)])


def _node_body(h_ref, agg_ref, w1h_ref, w1a_ref, bn1_ref, w2_ref, bn2_ref,
               o_ref):
    hx = h_ref[...]
    agg = agg_ref[0] + agg_ref[1]
    u = (jnp.dot(hx, w1h_ref[...], preferred_element_type=jnp.float32)
         + jnp.dot(agg, w1a_ref[...], preferred_element_type=jnp.float32)
         + bn1_ref[...])
    u = jnp.maximum(u, 0.0)
    o_ref[...] = (jnp.dot(u, w2_ref[...], preferred_element_type=jnp.float32)
                  + bn2_ref[...] + hx)


def _node_mlp(h, agg2, Wn1h, Wn1a, bn1, Wn2, bn2):
    BN = 400
    return pl.pallas_call(
        _node_body,
        grid=(N // BN,),
        in_specs=[
            pl.BlockSpec((BN, D), lambda i: (i, 0)),
            pl.BlockSpec((NC, BN, H), lambda i: (0, i, 0)),
            pl.BlockSpec((D, H), lambda i: (0, 0)),
            pl.BlockSpec((H, H), lambda i: (0, 0)),
            pl.BlockSpec((1, H), lambda i: (0, 0)),
            pl.BlockSpec((H, D), lambda i: (0, 0)),
            pl.BlockSpec((1, D), lambda i: (0, 0)),
        ],
        out_specs=pl.BlockSpec((BN, D), lambda i: (i, 0)),
        out_shape=jax.ShapeDtypeStruct((N, D), jnp.float32),
    )(h, agg2, Wn1h, Wn1a, bn1, Wn2, bn2)


def kernel(h, positions, edge_index, We1, be1, We2, be2, Wn1, bn1, Wn2, bn2):
    We1a = We1[:D]
    We1b = We1[D:2 * D]
    wr = We1[2 * D]
    px = positions[:, 0]
    py = positions[:, 1]
    pz = positions[:, 2]
    A, B = _precompute(h, We1a, We1b, be1.reshape(1, H))
    row = edge_index[0]
    col = edge_index[1]
    t_pre = _sc_edge(row, col, A, B, px, py, pz, wr)
    ef = _edge_mlp(t_pre, We2, be2.reshape(1, H))
    agg2 = _sc_scatter(row, ef)
    h_out = _node_mlp(h, agg2, Wn1[:D], Wn1[D:], bn1.reshape(1, H), Wn2,
                      bn2.reshape(1, D))
    return (h_out, positions)


# speedup vs baseline: 6.0573x; 1.3766x over previous
import functools

import jax
import jax.numpy as jnp
from jax import lax
from jax.experimental import pallas as pl
from jax.experimental.pallas import tpu as pltpu
from jax.experimental.pallas import tpu_sc as plsc

N = 10000
E = 320000
D = 128
H = 128

NC, NS, L = 2, 16, 16
NW = NC * NS
EPW = E // NW
C = 80
CHUNKS = EPW // C
GROUPS = C // L
ZB = N // NS // 5

_mesh = plsc.VectorSubcoreMesh(core_axis_name="c", subcore_axis_name="s")


def _pre_body(h_ref, wa_ref, wb_ref, be1_ref, a_ref, b_ref):
    x = h_ref[...]
    a_ref[...] = jnp.dot(x, wa_ref[...], preferred_element_type=jnp.float32)
    b_ref[...] = (jnp.dot(x, wb_ref[...], preferred_element_type=jnp.float32)
                  + be1_ref[...])


def _precompute(h, We1a, We1b, be1):
    BN = 1000
    return pl.pallas_call(
        _pre_body,
        grid=(N // BN,),
        in_specs=[
            pl.BlockSpec((BN, D), lambda i: (i, 0)),
            pl.BlockSpec((D, H), lambda i: (0, 0)),
            pl.BlockSpec((D, H), lambda i: (0, 0)),
            pl.BlockSpec((1, H), lambda i: (0, 0)),
        ],
        out_specs=[
            pl.BlockSpec((BN, H), lambda i: (i, 0)),
            pl.BlockSpec((BN, H), lambda i: (i, 0)),
        ],
        out_shape=[
            jax.ShapeDtypeStruct((N, H), jnp.float32),
            jax.ShapeDtypeStruct((N, H), jnp.float32),
        ],
    )(h, We1a, We1b, be1)


@functools.partial(
    pl.kernel,
    out_type=jax.ShapeDtypeStruct((E, H), jnp.float32),
    mesh=_mesh,
    scratch_types=[
        [pltpu.VMEM((C,), jnp.int32)] * 2,
        [pltpu.VMEM((C,), jnp.int32)] * 2,
        [pltpu.VMEM((C,), jnp.float32)] * 6,
        [pltpu.VMEM((C,), jnp.float32)] * 6,
        [pltpu.VMEM((C, H), jnp.float32)] * 2,
        [pltpu.VMEM((C, H), jnp.float32)] * 2,
        pltpu.VMEM((C + L,), jnp.float32),
        pltpu.VMEM((H,), jnp.float32),
        [pltpu.SemaphoreType.DMA] * 2,
        [pltpu.SemaphoreType.DMA] * 2,
        [pltpu.SemaphoreType.DMA] * 2,
    ],
)
def _sc_edge(row, col, A, B, px, py, pz, wr, t_out,
             row_v, col_v, pos0, pos1, a_v, b_v, rad_v, wr_v,
             s_ab, s_pos, s_out):
    wid = lax.axis_index("s") * NC + lax.axis_index("c")
    base0 = wid * EPW
    pltpu.sync_copy(wr, wr_v)
    wvecs = [wr_v[pl.ds(f * L, L)] for f in range(H // L)]
    pos = (pos0, pos1)

    def issue(ci, sl):
        base = base0 + ci * C
        pltpu.sync_copy(row.at[pl.ds(base, C)], row_v[sl])
        pltpu.sync_copy(col.at[pl.ds(base, C)], col_v[sl])
        pltpu.async_copy(A.at[row_v[sl]], a_v[sl], s_ab[sl])
        pltpu.async_copy(B.at[col_v[sl]], b_v[sl], s_ab[sl])
        pltpu.async_copy(px.at[row_v[sl]], pos[sl][0], s_pos[sl])
        pltpu.async_copy(py.at[row_v[sl]], pos[sl][1], s_pos[sl])
        pltpu.async_copy(pz.at[row_v[sl]], pos[sl][2], s_pos[sl])
        pltpu.async_copy(px.at[col_v[sl]], pos[sl][3], s_pos[sl])
        pltpu.async_copy(py.at[col_v[sl]], pos[sl][4], s_pos[sl])
        pltpu.async_copy(pz.at[col_v[sl]], pos[sl][5], s_pos[sl])

    def wait_gathers(sl):
        pltpu.make_async_copy(A.at[row_v[sl]], a_v[sl], s_ab[sl]).wait()
        pltpu.make_async_copy(B.at[col_v[sl]], b_v[sl], s_ab[sl]).wait()
        for j in range(6):
            pltpu.make_async_copy(px.at[row_v[sl]], pos[sl][j],
                                  s_pos[sl]).wait()

    def wait_out(ci, sl):
        base = base0 + ci * C
        pltpu.make_async_copy(a_v[sl], t_out.at[pl.ds(base, C)],
                              s_out[sl]).wait()

    issue(0, 0)

    def iteration(i, sl):
        q = 1 - sl

        @pl.when(i < CHUNKS - 1)
        def _prefetch():
            @pl.when(i >= 1)
            def _():
                wait_out(i - 1, q)

            issue(i + 1, q)

        wait_gathers(sl)
        for g in range(GROUPS):
            gsl = pl.ds(g * L, L)
            dx = pos[sl][0][gsl] - pos[sl][3][gsl]
            dy = pos[sl][1][gsl] - pos[sl][4][gsl]
            dz = pos[sl][2][gsl] - pos[sl][5][gsl]
            rad_v[gsl] = dx * dx + dy * dy + dz * dz

        def edge_body(e, carry2):
            rv = rad_v[pl.ds(e, L)]
            sp = jnp.full((L,), rv[0], jnp.float32)
            for f in range(H // L):
                sl2 = pl.ds(f * L, L)
                plsc.addupdate(a_v[sl].at[e, sl2],
                               b_v[sl][e, sl2] + sp * wvecs[f])
            return carry2

        lax.fori_loop(0, C, edge_body, 0, unroll=4)
        base = base0 + i * C
        pltpu.async_copy(a_v[sl], t_out.at[pl.ds(base, C)], s_out[sl])

    def chunk(i, carry):
        p = lax.rem(i, 2)

        @pl.when(p == 0)
        def _s0():
            iteration(i, 0)

        @pl.when(p == 1)
        def _s1():
            iteration(i, 1)

        return carry

    lax.fori_loop(0, CHUNKS, chunk, 0)
    wait_out(CHUNKS - 2, (CHUNKS - 2) % 2)
    wait_out(CHUNKS - 1, (CHUNKS - 1) % 2)


def _edge_mlp_body(t_ref, w2_ref, be2_ref, o_ref):
    t = jnp.maximum(t_ref[...], 0.0).astype(jnp.bfloat16)
    u = jnp.dot(t, w2_ref[...].astype(jnp.bfloat16),
                preferred_element_type=jnp.float32)
    o_ref[...] = jnp.maximum(u + be2_ref[...], 0.0)


def _edge_mlp(t_pre, We2, be2):
    BE = 2048
    return pl.pallas_call(
        _edge_mlp_body,
        grid=(E // BE,),
        in_specs=[
            pl.BlockSpec((BE, H), lambda i: (i, 0)),
            pl.BlockSpec((H, H), lambda i: (0, 0)),
            pl.BlockSpec((1, H), lambda i: (0, 0)),
        ],
        out_specs=pl.BlockSpec((BE, H), lambda i: (i, 0)),
        out_shape=jax.ShapeDtypeStruct((E, H), jnp.float32),
    )(t_pre, We2, be2)


@functools.partial(
    pl.kernel,
    out_type=jax.ShapeDtypeStruct((NC, N, H), jnp.float32),
    mesh=_mesh,
    scratch_types=[
        [pltpu.VMEM((C,), jnp.int32)] * 2,
        [pltpu.VMEM((C, H), jnp.float32)] * 2,
        pltpu.VMEM((ZB, H), jnp.float32),
        pltpu.VMEM_SHARED((N, H), jnp.float32),
        [pltpu.SemaphoreType.DMA] * 2,
        [pltpu.SemaphoreType.DMA] * 2,
    ],
)
def _sc_scatter(row, ef, out, row_v, ef_v, z_v, acc, s_ld, s_sc):
    cid = lax.axis_index("c")
    sid = lax.axis_index("s")
    wid = sid * NC + cid
    base0 = wid * EPW
    zeros = jnp.zeros((L,), jnp.float32)

    def zrow(r, carry):
        for f in range(H // L):
            z_v[r, pl.ds(f * L, L)] = zeros
        return carry

    lax.fori_loop(0, ZB, zrow, 0)
    rows_per_tile = N // NS
    for j in range(rows_per_tile // ZB):
        pltpu.sync_copy(z_v, acc.at[pl.ds(sid * rows_per_tile + j * ZB, ZB)])
    plsc.subcore_barrier()

    def issue(ci, sl):
        base = base0 + ci * C
        pltpu.sync_copy(row.at[pl.ds(base, C)], row_v[sl])
        pltpu.async_copy(ef.at[pl.ds(base, C)], ef_v[sl], s_ld[sl])

    def wait_ld(ci, sl):
        base = base0 + ci * C
        pltpu.make_async_copy(ef.at[pl.ds(base, C)], ef_v[sl],
                              s_ld[sl]).wait()

    def wait_sc(sl):
        pltpu.make_async_copy(ef_v[sl], acc.at[row_v[sl]], s_sc[sl]).wait()

    issue(0, 0)

    def iteration(i, sl):
        q = 1 - sl

        @pl.when(i < CHUNKS - 1)
        def _prefetch():
            @pl.when(i >= 2)
            def _():
                wait_sc(q)

            issue(i + 1, q)

        wait_ld(i, sl)
        pltpu.async_copy(ef_v[sl], acc.at[row_v[sl]], s_sc[sl], add=True)

    def chunk(i, carry):
        p = lax.rem(i, 2)

        @pl.when(p == 0)
        def _s0():
            iteration(i, 0)

        @pl.when(p == 1)
        def _s1():
            iteration(i, 1)

        return carry

    lax.fori_loop(0, CHUNKS, chunk, 0)
    wait_sc(0)
    wait_sc(1)
    plsc.subcore_barrier()
    RPT = 624
    start = sid * RPT
    pltpu.sync_copy(acc.at[pl.ds(start, RPT)], out.at[cid, pl.ds(start, RPT)])

    @pl.when(sid == NS - 1)
    def _tail():
        pltpu.sync_copy(acc.at[pl.ds(NS * RPT, N - NS * RPT)],
                        out.at[cid, pl.ds(NS * RPT, N - NS * RPT)])


def _node_body(h_ref, agg_ref, w1h_ref, w1a_ref, bn1_ref, w2_ref, bn2_ref,
               o_ref):
    hx = h_ref[...]
    agg = agg_ref[0] + agg_ref[1]
    u = (jnp.dot(hx, w1h_ref[...], preferred_element_type=jnp.float32)
         + jnp.dot(agg, w1a_ref[...], preferred_element_type=jnp.float32)
         + bn1_ref[...])
    u = jnp.maximum(u, 0.0)
    o_ref[...] = (jnp.dot(u, w2_ref[...], preferred_element_type=jnp.float32)
                  + bn2_ref[...] + hx)


def _node_mlp(h, agg2, Wn1h, Wn1a, bn1, Wn2, bn2):
    BN = 1000
    return pl.pallas_call(
        _node_body,
        grid=(N // BN,),
        in_specs=[
            pl.BlockSpec((BN, D), lambda i: (i, 0)),
            pl.BlockSpec((NC, BN, H), lambda i: (0, i, 0)),
            pl.BlockSpec((D, H), lambda i: (0, 0)),
            pl.BlockSpec((H, H), lambda i: (0, 0)),
            pl.BlockSpec((1, H), lambda i: (0, 0)),
            pl.BlockSpec((H, D), lambda i: (0, 0)),
            pl.BlockSpec((1, D), lambda i: (0, 0)),
        ],
        out_specs=pl.BlockSpec((BN, D), lambda i: (i, 0)),
        out_shape=jax.ShapeDtypeStruct((N, D), jnp.float32),
    )(h, agg2, Wn1h, Wn1a, bn1, Wn2, bn2)


def kernel(h, positions, edge_index, We1, be1, We2, be2, Wn1, bn1, Wn2, bn2):
    We1a = We1[:D]
    We1b = We1[D:2 * D]
    wr = We1[2 * D]
    px = positions[:, 0]
    py = positions[:, 1]
    pz = positions[:, 2]
    A, B = _precompute(h, We1a, We1b, be1.reshape(1, H))
    row = edge_index[0]
    col = edge_index[1]
    t_pre = _sc_edge(row, col, A, B, px, py, pz, wr)
    ef = _edge_mlp(t_pre, We2, be2.reshape(1, H))
    agg2 = _sc_scatter(row, ef)
    h_out = _node_mlp(h, agg2, Wn1[:D], Wn1[D:], bn1.reshape(1, H), Wn2,
                      bn2.reshape(1, D))
    return (h_out, positions)


# speedup vs baseline: 6.2396x; 1.0301x over previous
import functools

import jax
import jax.numpy as jnp
from jax import lax
from jax.experimental import pallas as pl
from jax.experimental.pallas import tpu as pltpu
from jax.experimental.pallas import tpu_sc as plsc

N = 10000
E = 320000
D = 128
H = 128

NC, NS, L = 2, 16, 16
NW = NC * NS
EPW = E // NW
C = 80
CHUNKS = EPW // C
GROUPS = C // L
ZB = N // NS // 5

_mesh = plsc.VectorSubcoreMesh(core_axis_name="c", subcore_axis_name="s")


def _pre_body(h_ref, wa_ref, wb_ref, be1_ref, a_ref, b_ref):
    x = h_ref[...]
    a_ref[...] = jnp.dot(x, wa_ref[...], preferred_element_type=jnp.float32)
    b_ref[...] = (jnp.dot(x, wb_ref[...], preferred_element_type=jnp.float32)
                  + be1_ref[...])


def _precompute(h, We1a, We1b, be1):
    BN = 1000
    return pl.pallas_call(
        _pre_body,
        grid=(N // BN,),
        in_specs=[
            pl.BlockSpec((BN, D), lambda i: (i, 0)),
            pl.BlockSpec((D, H), lambda i: (0, 0)),
            pl.BlockSpec((D, H), lambda i: (0, 0)),
            pl.BlockSpec((1, H), lambda i: (0, 0)),
        ],
        out_specs=[
            pl.BlockSpec((BN, H), lambda i: (i, 0)),
            pl.BlockSpec((BN, H), lambda i: (i, 0)),
        ],
        out_shape=[
            jax.ShapeDtypeStruct((N, H), jnp.float32),
            jax.ShapeDtypeStruct((N, H), jnp.float32),
        ],
    )(h, We1a, We1b, be1)


@functools.partial(
    pl.kernel,
    out_type=jax.ShapeDtypeStruct((E, H), jnp.float32),
    mesh=_mesh,
    scratch_types=[
        [pltpu.VMEM((C,), jnp.int32)] * 2,
        [pltpu.VMEM((C,), jnp.int32)] * 2,
        [pltpu.VMEM((C,), jnp.float32)] * 6,
        [pltpu.VMEM((C,), jnp.float32)] * 6,
        [pltpu.VMEM((C, H), jnp.float32)] * 2,
        [pltpu.VMEM((C, H), jnp.float32)] * 2,
        pltpu.VMEM((C + L,), jnp.float32),
        pltpu.VMEM((H,), jnp.float32),
        [pltpu.SemaphoreType.DMA] * 2,
        [pltpu.SemaphoreType.DMA] * 2,
        [pltpu.SemaphoreType.DMA] * 2,
    ],
)
def _sc_edge(row, col, A, B, px, py, pz, wr, t_out,
             row_v, col_v, pos0, pos1, a_v, b_v, rad_v, wr_v,
             s_ab, s_pos, s_out):
    wid = lax.axis_index("s") * NC + lax.axis_index("c")
    base0 = wid * EPW
    pltpu.sync_copy(wr, wr_v)
    wvecs = [wr_v[pl.ds(f * L, L)] for f in range(H // L)]
    pos = (pos0, pos1)

    def issue(ci, sl):
        base = base0 + ci * C
        pltpu.sync_copy(row.at[pl.ds(base, C)], row_v[sl])
        pltpu.sync_copy(col.at[pl.ds(base, C)], col_v[sl])
        pltpu.async_copy(A.at[row_v[sl]], a_v[sl], s_ab[sl])
        pltpu.async_copy(B.at[col_v[sl]], b_v[sl], s_ab[sl])
        pltpu.async_copy(px.at[row_v[sl]], pos[sl][0], s_pos[sl])
        pltpu.async_copy(py.at[row_v[sl]], pos[sl][1], s_pos[sl])
        pltpu.async_copy(pz.at[row_v[sl]], pos[sl][2], s_pos[sl])
        pltpu.async_copy(px.at[col_v[sl]], pos[sl][3], s_pos[sl])
        pltpu.async_copy(py.at[col_v[sl]], pos[sl][4], s_pos[sl])
        pltpu.async_copy(pz.at[col_v[sl]], pos[sl][5], s_pos[sl])

    def wait_gathers(sl):
        pltpu.make_async_copy(A.at[row_v[sl]], a_v[sl], s_ab[sl]).wait()
        pltpu.make_async_copy(B.at[col_v[sl]], b_v[sl], s_ab[sl]).wait()
        for j in range(6):
            pltpu.make_async_copy(px.at[row_v[sl]], pos[sl][j],
                                  s_pos[sl]).wait()

    def wait_out(ci, sl):
        base = base0 + ci * C
        pltpu.make_async_copy(a_v[sl], t_out.at[pl.ds(base, C)],
                              s_out[sl]).wait()

    issue(0, 0)

    def iteration(i, sl):
        q = 1 - sl

        @pl.when(i < CHUNKS - 1)
        def _prefetch():
            @pl.when(i >= 1)
            def _():
                wait_out(i - 1, q)

            issue(i + 1, q)

        wait_gathers(sl)
        for g in range(GROUPS):
            gsl = pl.ds(g * L, L)
            dx = pos[sl][0][gsl] - pos[sl][3][gsl]
            dy = pos[sl][1][gsl] - pos[sl][4][gsl]
            dz = pos[sl][2][gsl] - pos[sl][5][gsl]
            rad_v[gsl] = dx * dx + dy * dy + dz * dz

        def edge_body(e, carry2):
            rv = rad_v[pl.ds(e, L)]
            sp = jnp.full((L,), rv[0], jnp.float32)
            for f in range(H // L):
                sl2 = pl.ds(f * L, L)
                plsc.addupdate(a_v[sl].at[e, sl2],
                               b_v[sl][e, sl2] + sp * wvecs[f])
            return carry2

        lax.fori_loop(0, C, edge_body, 0, unroll=4)
        base = base0 + i * C
        pltpu.async_copy(a_v[sl], t_out.at[pl.ds(base, C)], s_out[sl])

    def chunk(i, carry):
        p = lax.rem(i, 2)

        @pl.when(p == 0)
        def _s0():
            iteration(i, 0)

        @pl.when(p == 1)
        def _s1():
            iteration(i, 1)

        return carry

    lax.fori_loop(0, CHUNKS, chunk, 0)
    wait_out(CHUNKS - 2, (CHUNKS - 2) % 2)
    wait_out(CHUNKS - 1, (CHUNKS - 1) % 2)


def _edge_mlp_body(t_ref, w2_ref, be2_ref, o_ref):
    t = jnp.maximum(t_ref[...], 0.0).astype(jnp.bfloat16)
    u = jnp.dot(t, w2_ref[...].astype(jnp.bfloat16),
                preferred_element_type=jnp.float32)
    o_ref[...] = jnp.maximum(u + be2_ref[...], 0.0)


def _edge_mlp(t_pre, We2, be2):
    BE = 2560
    return pl.pallas_call(
        _edge_mlp_body,
        grid=(E // BE,),
        in_specs=[
            pl.BlockSpec((BE, H), lambda i: (i, 0)),
            pl.BlockSpec((H, H), lambda i: (0, 0)),
            pl.BlockSpec((1, H), lambda i: (0, 0)),
        ],
        out_specs=pl.BlockSpec((BE, H), lambda i: (i, 0)),
        out_shape=jax.ShapeDtypeStruct((E, H), jnp.float32),
    )(t_pre, We2, be2)


@functools.partial(
    pl.kernel,
    out_type=jax.ShapeDtypeStruct((NC, N, H), jnp.float32),
    mesh=_mesh,
    scratch_types=[
        [pltpu.VMEM((C,), jnp.int32)] * 2,
        [pltpu.VMEM((C, H), jnp.float32)] * 2,
        pltpu.VMEM((ZB, H), jnp.float32),
        pltpu.VMEM_SHARED((N, H), jnp.float32),
        [pltpu.SemaphoreType.DMA] * 2,
        [pltpu.SemaphoreType.DMA] * 2,
    ],
)
def _sc_scatter(row, ef, out, row_v, ef_v, z_v, acc, s_ld, s_sc):
    cid = lax.axis_index("c")
    sid = lax.axis_index("s")
    wid = sid * NC + cid
    base0 = wid * EPW
    zeros = jnp.zeros((L,), jnp.float32)

    def zrow(r, carry):
        for f in range(H // L):
            z_v[r, pl.ds(f * L, L)] = zeros
        return carry

    lax.fori_loop(0, ZB, zrow, 0)
    rows_per_tile = N // NS
    for j in range(rows_per_tile // ZB):
        pltpu.sync_copy(z_v, acc.at[pl.ds(sid * rows_per_tile + j * ZB, ZB)])
    plsc.subcore_barrier()

    def issue(ci, sl):
        base = base0 + ci * C
        pltpu.sync_copy(row.at[pl.ds(base, C)], row_v[sl])
        pltpu.async_copy(ef.at[pl.ds(base, C)], ef_v[sl], s_ld[sl])

    def wait_ld(ci, sl):
        base = base0 + ci * C
        pltpu.make_async_copy(ef.at[pl.ds(base, C)], ef_v[sl],
                              s_ld[sl]).wait()

    def wait_sc(sl):
        pltpu.make_async_copy(ef_v[sl], acc.at[row_v[sl]], s_sc[sl]).wait()

    issue(0, 0)

    def iteration(i, sl):
        q = 1 - sl

        @pl.when(i < CHUNKS - 1)
        def _prefetch():
            @pl.when(i >= 2)
            def _():
                wait_sc(q)

            issue(i + 1, q)

        wait_ld(i, sl)
        pltpu.async_copy(ef_v[sl], acc.at[row_v[sl]], s_sc[sl], add=True)

    def chunk(i, carry):
        p = lax.rem(i, 2)

        @pl.when(p == 0)
        def _s0():
            iteration(i, 0)

        @pl.when(p == 1)
        def _s1():
            iteration(i, 1)

        return carry

    lax.fori_loop(0, CHUNKS, chunk, 0)
    wait_sc(0)
    wait_sc(1)
    plsc.subcore_barrier()
    RPT = 624
    start = sid * RPT
    pltpu.sync_copy(acc.at[pl.ds(start, RPT)], out.at[cid, pl.ds(start, RPT)])

    @pl.when(sid == NS - 1)
    def _tail():
        pltpu.sync_copy(acc.at[pl.ds(NS * RPT, N - NS * RPT)],
                        out.at[cid, pl.ds(NS * RPT, N - NS * RPT)])


def _node_body(h_ref, agg_ref, w1h_ref, w1a_ref, bn1_ref, w2_ref, bn2_ref,
               o_ref):
    hx = h_ref[...]
    agg = agg_ref[0] + agg_ref[1]
    u = (jnp.dot(hx, w1h_ref[...], preferred_element_type=jnp.float32)
         + jnp.dot(agg, w1a_ref[...], preferred_element_type=jnp.float32)
         + bn1_ref[...])
    u = jnp.maximum(u, 0.0)
    o_ref[...] = (jnp.dot(u, w2_ref[...], preferred_element_type=jnp.float32)
                  + bn2_ref[...] + hx)


def _node_mlp(h, agg2, Wn1h, Wn1a, bn1, Wn2, bn2):
    BN = 1000
    return pl.pallas_call(
        _node_body,
        grid=(N // BN,),
        in_specs=[
            pl.BlockSpec((BN, D), lambda i: (i, 0)),
            pl.BlockSpec((NC, BN, H), lambda i: (0, i, 0)),
            pl.BlockSpec((D, H), lambda i: (0, 0)),
            pl.BlockSpec((H, H), lambda i: (0, 0)),
            pl.BlockSpec((1, H), lambda i: (0, 0)),
            pl.BlockSpec((H, D), lambda i: (0, 0)),
            pl.BlockSpec((1, D), lambda i: (0, 0)),
        ],
        out_specs=pl.BlockSpec((BN, D), lambda i: (i, 0)),
        out_shape=jax.ShapeDtypeStruct((N, D), jnp.float32),
    )(h, agg2, Wn1h, Wn1a, bn1, Wn2, bn2)


def kernel(h, positions, edge_index, We1, be1, We2, be2, Wn1, bn1, Wn2, bn2):
    We1a = We1[:D]
    We1b = We1[D:2 * D]
    wr = We1[2 * D]
    px = positions[:, 0]
    py = positions[:, 1]
    pz = positions[:, 2]
    A, B = _precompute(h, We1a, We1b, be1.reshape(1, H))
    row = edge_index[0]
    col = edge_index[1]
    t_pre = _sc_edge(row, col, A, B, px, py, pz, wr)
    ef = _edge_mlp(t_pre, We2, be2.reshape(1, H))
    agg2 = _sc_scatter(row, ef)
    h_out = _node_mlp(h, agg2, Wn1[:D], Wn1[D:], bn1.reshape(1, H), Wn2,
                      bn2.reshape(1, D))
    return (h_out, positions)


# speedup vs baseline: 6.4099x; 1.0273x over previous
import functools

import jax
import jax.numpy as jnp
from jax import lax
from jax.experimental import pallas as pl
from jax.experimental.pallas import tpu as pltpu
from jax.experimental.pallas import tpu_sc as plsc

N = 10000
E = 320000
D = 128
H = 128

NC, NS, L = 2, 16, 16
NW = NC * NS
S = 5
ES = E // S
EPS = ES // NW
C = 80
SCHUNKS = EPS // C
GROUPS = C // L
ZB = N // NS // 5

_mesh = plsc.VectorSubcoreMesh(core_axis_name="c", subcore_axis_name="s")


def _pre_body(h_ref, wa_ref, wb_ref, be1_ref, a_ref, b_ref):
    x = h_ref[...]
    a_ref[...] = jnp.dot(x, wa_ref[...], preferred_element_type=jnp.float32)
    b_ref[...] = (jnp.dot(x, wb_ref[...], preferred_element_type=jnp.float32)
                  + be1_ref[...])


def _precompute(h, We1a, We1b, be1):
    BN = 1000
    return pl.pallas_call(
        _pre_body,
        grid=(N // BN,),
        in_specs=[
            pl.BlockSpec((BN, D), lambda i: (i, 0)),
            pl.BlockSpec((D, H), lambda i: (0, 0)),
            pl.BlockSpec((D, H), lambda i: (0, 0)),
            pl.BlockSpec((1, H), lambda i: (0, 0)),
        ],
        out_specs=[
            pl.BlockSpec((BN, H), lambda i: (i, 0)),
            pl.BlockSpec((BN, H), lambda i: (i, 0)),
        ],
        out_shape=[
            jax.ShapeDtypeStruct((N, H), jnp.float32),
            jax.ShapeDtypeStruct((N, H), jnp.float32),
        ],
    )(h, We1a, We1b, be1)


def _make_sc_edge(sbase):
    @functools.partial(
        pl.kernel,
        out_type=jax.ShapeDtypeStruct((ES, H), jnp.float32),
        mesh=_mesh,
        scratch_types=[
            [pltpu.VMEM((C,), jnp.int32)] * 2,
            [pltpu.VMEM((C,), jnp.int32)] * 2,
            [pltpu.VMEM((C,), jnp.float32)] * 6,
            [pltpu.VMEM((C,), jnp.float32)] * 6,
            [pltpu.VMEM((C, H), jnp.float32)] * 2,
            [pltpu.VMEM((C, H), jnp.float32)] * 2,
            pltpu.VMEM((C + L,), jnp.float32),
            pltpu.VMEM((H,), jnp.float32),
            [pltpu.SemaphoreType.DMA] * 2,
            [pltpu.SemaphoreType.DMA] * 2,
            [pltpu.SemaphoreType.DMA] * 2,
        ],
    )
    def _sc_edge(row, col, A, B, px, py, pz, wr, t_out,
                 row_v, col_v, pos0, pos1, a_v, b_v, rad_v, wr_v,
                 s_ab, s_pos, s_out):
        wid = lax.axis_index("s") * NC + lax.axis_index("c")
        gbase = sbase + wid * EPS
        obase = wid * EPS
        pltpu.sync_copy(wr, wr_v)
        wvecs = [wr_v[pl.ds(f * L, L)] for f in range(H // L)]
        pos = (pos0, pos1)

        def issue(ci, sl):
            base = gbase + ci * C
            pltpu.sync_copy(row.at[pl.ds(base, C)], row_v[sl])
            pltpu.sync_copy(col.at[pl.ds(base, C)], col_v[sl])
            pltpu.async_copy(A.at[row_v[sl]], a_v[sl], s_ab[sl])
            pltpu.async_copy(B.at[col_v[sl]], b_v[sl], s_ab[sl])
            pltpu.async_copy(px.at[row_v[sl]], pos[sl][0], s_pos[sl])
            pltpu.async_copy(py.at[row_v[sl]], pos[sl][1], s_pos[sl])
            pltpu.async_copy(pz.at[row_v[sl]], pos[sl][2], s_pos[sl])
            pltpu.async_copy(px.at[col_v[sl]], pos[sl][3], s_pos[sl])
            pltpu.async_copy(py.at[col_v[sl]], pos[sl][4], s_pos[sl])
            pltpu.async_copy(pz.at[col_v[sl]], pos[sl][5], s_pos[sl])

        def wait_gathers(sl):
            pltpu.make_async_copy(A.at[row_v[sl]], a_v[sl], s_ab[sl]).wait()
            pltpu.make_async_copy(B.at[col_v[sl]], b_v[sl], s_ab[sl]).wait()
            for j in range(6):
                pltpu.make_async_copy(px.at[row_v[sl]], pos[sl][j],
                                      s_pos[sl]).wait()

        def wait_out(ci, sl):
            base = obase + ci * C
            pltpu.make_async_copy(a_v[sl], t_out.at[pl.ds(base, C)],
                                  s_out[sl]).wait()

        issue(0, 0)

        def iteration(i, sl):
            q = 1 - sl

            @pl.when(i < SCHUNKS - 1)
            def _prefetch():
                @pl.when(i >= 1)
                def _():
                    wait_out(i - 1, q)

                issue(i + 1, q)

            wait_gathers(sl)
            for g in range(GROUPS):
                gsl = pl.ds(g * L, L)
                dx = pos[sl][0][gsl] - pos[sl][3][gsl]
                dy = pos[sl][1][gsl] - pos[sl][4][gsl]
                dz = pos[sl][2][gsl] - pos[sl][5][gsl]
                rad_v[gsl] = dx * dx + dy * dy + dz * dz

            def edge_body(e, carry2):
                rv = rad_v[pl.ds(e, L)]
                sp = jnp.full((L,), rv[0], jnp.float32)
                for f in range(H // L):
                    sl2 = pl.ds(f * L, L)
                    plsc.addupdate(a_v[sl].at[e, sl2],
                                   b_v[sl][e, sl2] + sp * wvecs[f])
                return carry2

            lax.fori_loop(0, C, edge_body, 0, unroll=4)
            base = obase + i * C
            pltpu.async_copy(a_v[sl], t_out.at[pl.ds(base, C)], s_out[sl])

        def chunk(i, carry):
            p = lax.rem(i, 2)

            @pl.when(p == 0)
            def _s0():
                iteration(i, 0)

            @pl.when(p == 1)
            def _s1():
                iteration(i, 1)

            return carry

        lax.fori_loop(0, SCHUNKS, chunk, 0)
        wait_out(SCHUNKS - 2, (SCHUNKS - 2) % 2)
        wait_out(SCHUNKS - 1, (SCHUNKS - 1) % 2)

    return _sc_edge


_sc_edge_slices = [_make_sc_edge(s * ES) for s in range(S)]


def _edge_mlp_body(t_ref, w2_ref, be2_ref, o_ref):
    t = jnp.maximum(t_ref[...], 0.0).astype(jnp.bfloat16)
    u = jnp.dot(t, w2_ref[...].astype(jnp.bfloat16),
                preferred_element_type=jnp.float32)
    o_ref[...] = jnp.maximum(u + be2_ref[...], 0.0)


def _edge_mlp(t_pre, We2, be2):
    BE = 2000
    return pl.pallas_call(
        _edge_mlp_body,
        grid=(ES // BE,),
        in_specs=[
            pl.BlockSpec((BE, H), lambda i: (i, 0)),
            pl.BlockSpec((H, H), lambda i: (0, 0)),
            pl.BlockSpec((1, H), lambda i: (0, 0)),
        ],
        out_specs=pl.BlockSpec((BE, H), lambda i: (i, 0)),
        out_shape=jax.ShapeDtypeStruct((ES, H), jnp.float32),
    )(t_pre, We2, be2)


@functools.partial(
    pl.kernel,
    out_type=jax.ShapeDtypeStruct((NC, N, H), jnp.float32),
    mesh=_mesh,
    scratch_types=[
        [pltpu.VMEM((C,), jnp.int32)] * 2,
        [pltpu.VMEM((C, H), jnp.float32)] * 2,
        pltpu.VMEM((ZB, H), jnp.float32),
        pltpu.VMEM_SHARED((N, H), jnp.float32),
        [pltpu.SemaphoreType.DMA] * 2,
        [pltpu.SemaphoreType.DMA] * 2,
    ],
)
def _sc_scatter(row, ef0, ef1, ef2, ef3, ef4, out,
                row_v, ef_v, z_v, acc, s_ld, s_sc):
    cid = lax.axis_index("c")
    sid = lax.axis_index("s")
    wid = sid * NC + cid
    zeros = jnp.zeros((L,), jnp.float32)

    def zrow(r, carry):
        for f in range(H // L):
            z_v[r, pl.ds(f * L, L)] = zeros
        return carry

    lax.fori_loop(0, ZB, zrow, 0)
    rows_per_tile = N // NS
    for j in range(rows_per_tile // ZB):
        pltpu.sync_copy(z_v, acc.at[pl.ds(sid * rows_per_tile + j * ZB, ZB)])
    plsc.subcore_barrier()

    def run_slice(ef, sbase):
        gbase = sbase + wid * EPS
        ebase = wid * EPS

        def issue(ci, sl):
            pltpu.sync_copy(row.at[pl.ds(gbase + ci * C, C)], row_v[sl])
            pltpu.async_copy(ef.at[pl.ds(ebase + ci * C, C)], ef_v[sl],
                             s_ld[sl])

        def wait_ld(ci, sl):
            pltpu.make_async_copy(ef.at[pl.ds(ebase + ci * C, C)], ef_v[sl],
                                  s_ld[sl]).wait()

        def wait_sc(sl):
            pltpu.make_async_copy(ef_v[sl], acc.at[row_v[sl]],
                                  s_sc[sl]).wait()

        issue(0, 0)

        def iteration(i, sl):
            q = 1 - sl

            @pl.when(i < SCHUNKS - 1)
            def _prefetch():
                @pl.when(i >= 2)
                def _():
                    wait_sc(q)

                issue(i + 1, q)

            wait_ld(i, sl)
            pltpu.async_copy(ef_v[sl], acc.at[row_v[sl]], s_sc[sl], add=True)

        def chunk(i, carry):
            p = lax.rem(i, 2)

            @pl.when(p == 0)
            def _s0():
                iteration(i, 0)

            @pl.when(p == 1)
            def _s1():
                iteration(i, 1)

            return carry

        lax.fori_loop(0, SCHUNKS, chunk, 0)
        wait_sc(0)
        wait_sc(1)

    for s_i, ef in enumerate((ef0, ef1, ef2, ef3, ef4)):
        run_slice(ef, s_i * ES)
    plsc.subcore_barrier()
    RPT = 624
    start = sid * RPT
    pltpu.sync_copy(acc.at[pl.ds(start, RPT)], out.at[cid, pl.ds(start, RPT)])

    @pl.when(sid == NS - 1)
    def _tail():
        pltpu.sync_copy(acc.at[pl.ds(NS * RPT, N - NS * RPT)],
                        out.at[cid, pl.ds(NS * RPT, N - NS * RPT)])


def _node_body(h_ref, agg_ref, w1h_ref, w1a_ref, bn1_ref, w2_ref, bn2_ref,
               o_ref):
    hx = h_ref[...]
    agg = agg_ref[0] + agg_ref[1]
    u = (jnp.dot(hx, w1h_ref[...], preferred_element_type=jnp.float32)
         + jnp.dot(agg, w1a_ref[...], preferred_element_type=jnp.float32)
         + bn1_ref[...])
    u = jnp.maximum(u, 0.0)
    o_ref[...] = (jnp.dot(u, w2_ref[...], preferred_element_type=jnp.float32)
                  + bn2_ref[...] + hx)


def _node_mlp(h, agg2, Wn1h, Wn1a, bn1, Wn2, bn2):
    BN = 1000
    return pl.pallas_call(
        _node_body,
        grid=(N // BN,),
        in_specs=[
            pl.BlockSpec((BN, D), lambda i: (i, 0)),
            pl.BlockSpec((NC, BN, H), lambda i: (0, i, 0)),
            pl.BlockSpec((D, H), lambda i: (0, 0)),
            pl.BlockSpec((H, H), lambda i: (0, 0)),
            pl.BlockSpec((1, H), lambda i: (0, 0)),
            pl.BlockSpec((H, D), lambda i: (0, 0)),
            pl.BlockSpec((1, D), lambda i: (0, 0)),
        ],
        out_specs=pl.BlockSpec((BN, D), lambda i: (i, 0)),
        out_shape=jax.ShapeDtypeStruct((N, D), jnp.float32),
    )(h, agg2, Wn1h, Wn1a, bn1, Wn2, bn2)


def kernel(h, positions, edge_index, We1, be1, We2, be2, Wn1, bn1, Wn2, bn2):
    We1a = We1[:D]
    We1b = We1[D:2 * D]
    wr = We1[2 * D]
    px = positions[:, 0]
    py = positions[:, 1]
    pz = positions[:, 2]
    A, B = _precompute(h, We1a, We1b, be1.reshape(1, H))
    row = edge_index[0]
    col = edge_index[1]
    efs = []
    for s_i in range(S):
        t_pre = _sc_edge_slices[s_i](row, col, A, B, px, py, pz, wr)
        efs.append(_edge_mlp(t_pre, We2, be2.reshape(1, H)))
    agg2 = _sc_scatter(row, *efs)
    h_out = _node_mlp(h, agg2, Wn1[:D], Wn1[D:], bn1.reshape(1, H), Wn2,
                      bn2.reshape(1, D))
    return (h_out, positions)
